# Initial kernel scaffold; baseline (speedup 1.0000x reference)
import jax
import jax.numpy as jnp
from jax.experimental import pallas as pl


def kernel(x, Wg, bg, W1, b1, W2, b2, W3, b3):
    raise NotImplementedError("write your pallas kernel here")


# speedup vs baseline: 2.1981x; 2.1981x over previous
import functools

import jax
import jax.numpy as jnp
from jax import lax
from jax.experimental import pallas as pl
from jax.experimental.pallas import tpu as pltpu
from jax.experimental.pallas import tpu_sc as plsc

DIM = 1024
E = 8
HID = 4096
S = 4096
BLK_M = 256
NT = 40
N = NT * BLK_M
NW = 32
TOK_PER_W = S // NW
CH = 32


def _shift_down(a, k):
    return jnp.concatenate([jnp.zeros((k,) + a.shape[1:], a.dtype), a[:-k]], axis=0)


def _incl_cumsum0(a):
    acc = a
    k = 1
    while k < a.shape[0]:
        acc = acc + _shift_down(acc, k)
        k *= 2
    return acc


def _router_kernel(xf_ref, wg_ref, bg_ref, aux_ref, mi_ref, g0r_ref, g1r_ref):
    xf = xf_ref[...]
    wg = wg_ref[...]
    logits = lax.dot_general(
        xf, wg, (((1,), (0,)), ((), ())),
        precision=lax.Precision.DEFAULT,
        preferred_element_type=jnp.float32) + bg_ref[...]
    m = jnp.max(logits, axis=1, keepdims=True)
    p = jnp.exp(logits - m)
    probs = p / jnp.sum(p, axis=1, keepdims=True)

    eidx = lax.broadcasted_iota(jnp.int32, (S, E), 1)
    v0 = jnp.max(probs, axis=1, keepdims=True)
    i0 = jnp.min(jnp.where(probs == v0, eidx, E), axis=1, keepdims=True)
    oh0 = (eidx == i0).astype(jnp.float32)
    probs_m = jnp.where(eidx == i0, -1.0, probs)
    v1 = jnp.max(probs_m, axis=1, keepdims=True)
    i1 = jnp.min(jnp.where(probs_m == v1, eidx, E), axis=1, keepdims=True)
    oh1 = (eidx == i1).astype(jnp.float32)

    importance = jnp.mean(probs, axis=0, keepdims=True)
    load = jnp.sum(oh0, axis=0, keepdims=True) / float(S)
    aux_ref[...] = jnp.reshape(float(E) * jnp.sum(importance * load), (1, 1))

    both = oh0 + oh1
    c_in = _incl_cumsum0(both)
    c_ex = c_in - both
    rank0 = jnp.sum(c_ex * oh0, axis=1)
    rank1 = jnp.sum((c_ex + oh0) * oh1, axis=1)

    counts = c_in[-1:, :]
    cap = jnp.ceil(counts / float(BLK_M)) * float(BLK_M)
    base = cap
    k = 1
    while k < E:
        base = base + jnp.concatenate(
            [jnp.zeros((1, k), jnp.float32), base[:, :-k]], axis=1)
        k *= 2
    base_ex = base - cap
    ends = base_ex + cap

    pos0 = jnp.sum(base_ex * oh0, axis=1) + rank0
    pos1 = jnp.sum(base_ex * oh1, axis=1) + rank1

    tstart = lax.broadcasted_iota(jnp.int32, (S, 1), 0).astype(
        jnp.float32) * float(BLK_M)
    te = jnp.sum((ends <= tstart).astype(jnp.float32), axis=1)
    te = jnp.minimum(te, float(E - 1))

    mi_ref[...] = jnp.concatenate(
        [pos0[None, :], pos1[None, :], te[None, :],
         jnp.zeros((5, S), jnp.float32)], axis=0).astype(jnp.int32)
    g0r_ref[...] = jnp.broadcast_to(v0, (S, 16))
    g1r_ref[...] = jnp.broadcast_to(v1, (S, 16))


def _route(xf, Wg, bg):
    return pl.pallas_call(
        _router_kernel,
        out_shape=[
            jax.ShapeDtypeStruct((1, 1), jnp.float32),
            jax.ShapeDtypeStruct((8, S), jnp.int32),
            jax.ShapeDtypeStruct((S, 16), jnp.float32),
            jax.ShapeDtypeStruct((S, 16), jnp.float32),
        ],
    )(xf, Wg, bg.reshape(1, E))


def _gemm_kernel(te_ref, xs_ref, w1_ref, b1_ref, w2_ref, b2_ref, w3_ref,
                 b3_ref, out_ref):
    xb = xs_ref[...].astype(jnp.bfloat16)
    a = lax.dot_general(xb, w1_ref[0], (((1,), (0,)), ((), ())),
                        preferred_element_type=jnp.float32) + b1_ref[0]
    b = lax.dot_general(xb, w2_ref[0], (((1,), (0,)), ((), ())),
                        preferred_element_type=jnp.float32) + b2_ref[0]
    h = (a * jax.nn.sigmoid(a) * b).astype(jnp.bfloat16)
    out_ref[...] = lax.dot_general(
        h, w3_ref[0], (((1,), (0,)), ((), ())),
        preferred_element_type=jnp.float32) + b3_ref[0]


def _gemm(te, xs, W1, b1, W2, b2, W3, b3):
    grid_spec = pltpu.PrefetchScalarGridSpec(
        num_scalar_prefetch=1,
        grid=(NT,),
        in_specs=[
            pl.BlockSpec((BLK_M, DIM), lambda i, te: (i, 0)),
            pl.BlockSpec((1, DIM, HID), lambda i, te: (te[i], 0, 0)),
            pl.BlockSpec((1, 1, HID), lambda i, te: (te[i], 0, 0)),
            pl.BlockSpec((1, DIM, HID), lambda i, te: (te[i], 0, 0)),
            pl.BlockSpec((1, 1, HID), lambda i, te: (te[i], 0, 0)),
            pl.BlockSpec((1, HID, DIM), lambda i, te: (te[i], 0, 0)),
            pl.BlockSpec((1, 1, DIM), lambda i, te: (te[i], 0, 0)),
        ],
        out_specs=pl.BlockSpec((BLK_M, DIM), lambda i, te: (i, 0)),
    )
    return pl.pallas_call(
        _gemm_kernel,
        grid_spec=grid_spec,
        out_shape=jax.ShapeDtypeStruct((N, DIM), jnp.float32),
    )(te, xs, W1, b1, W2, b2, W3, b3)


def _dispatch_body(xf_hbm, pos0_hbm, pos1_hbm, xs_hbm, rows_v, idx0_v, idx1_v,
                   sem0, sem1):
    wid = lax.axis_index("s") * 2 + lax.axis_index("c")
    for c in range(TOK_PER_W // CH):
        base = wid * TOK_PER_W + c * CH
        pltpu.sync_copy(pos0_hbm.at[pl.ds(base, CH)], idx0_v)
        pltpu.sync_copy(pos1_hbm.at[pl.ds(base, CH)], idx1_v)
        pltpu.sync_copy(xf_hbm.at[pl.ds(base, CH)], rows_v)
        cp0 = pltpu.async_copy(rows_v, xs_hbm.at[idx0_v], sem0)
        cp1 = pltpu.async_copy(rows_v, xs_hbm.at[idx1_v], sem1)
        cp0.wait()
        cp1.wait()


def _combine_body(out_hbm, pos0_hbm, pos1_hbm, g0_hbm, g1_hbm, y_hbm,
                  b0_v, b1_v, y_v, idx0_v, idx1_v, g0_v, g1_v, sem0, sem1):
    wid = lax.axis_index("s") * 2 + lax.axis_index("c")
    for c in range(TOK_PER_W // CH):
        base = wid * TOK_PER_W + c * CH
        pltpu.sync_copy(pos0_hbm.at[pl.ds(base, CH)], idx0_v)
        pltpu.sync_copy(pos1_hbm.at[pl.ds(base, CH)], idx1_v)
        pltpu.sync_copy(g0_hbm.at[pl.ds(base, CH)], g0_v)
        pltpu.sync_copy(g1_hbm.at[pl.ds(base, CH)], g1_v)
        cp0 = pltpu.async_copy(out_hbm.at[idx0_v], b0_v, sem0)
        cp1 = pltpu.async_copy(out_hbm.at[idx1_v], b1_v, sem1)
        cp0.wait()
        cp1.wait()

        def body(r, _):
            g0s = g0_v[r]
            g1s = g1_v[r]
            for j in range(DIM // 16):
                sl = pl.ds(j * 16, 16)
                y_v[r, sl] = g0s * b0_v[r, sl] + g1s * b1_v[r, sl]
            return 0

        lax.fori_loop(0, CH, body, 0)
        pltpu.sync_copy(y_v, y_hbm.at[pl.ds(base, CH)])


@functools.lru_cache(maxsize=1)
def _sc_kernels():
    mesh = plsc.VectorSubcoreMesh(core_axis_name="c", subcore_axis_name="s")
    dispatch = pl.kernel(
        _dispatch_body,
        out_type=jax.ShapeDtypeStruct((N, DIM), jnp.float32),
        mesh=mesh,
        scratch_types=[
            pltpu.VMEM((CH, DIM), jnp.float32),
            pltpu.VMEM((CH,), jnp.int32),
            pltpu.VMEM((CH,), jnp.int32),
            pltpu.SemaphoreType.DMA,
            pltpu.SemaphoreType.DMA,
        ],
    )
    combine = pl.kernel(
        _combine_body,
        out_type=jax.ShapeDtypeStruct((S, DIM), jnp.float32),
        mesh=mesh,
        scratch_types=[
            pltpu.VMEM((CH, DIM), jnp.float32),
            pltpu.VMEM((CH, DIM), jnp.float32),
            pltpu.VMEM((CH, DIM), jnp.float32),
            pltpu.VMEM((CH,), jnp.int32),
            pltpu.VMEM((CH,), jnp.int32),
            pltpu.VMEM((CH, 16), jnp.float32),
            pltpu.VMEM((CH, 16), jnp.float32),
            pltpu.SemaphoreType.DMA,
            pltpu.SemaphoreType.DMA,
        ],
    )
    return dispatch, combine


def kernel(x, Wg, bg, W1, b1, W2, b2, W3, b3):
    Bb, Tt, C = x.shape
    xf = x.reshape(S, C)
    aux_a, mi, g0r, g1r = _route(xf, Wg, bg)
    pos0, pos1, te = mi[0], mi[1], mi[2, :NT]
    _dispatch, _combine = _sc_kernels()
    xs = _dispatch(xf, pos0, pos1)
    out = _gemm(te, xs,
                W1.astype(jnp.bfloat16), b1.reshape(E, 1, HID),
                W2.astype(jnp.bfloat16), b2.reshape(E, 1, HID),
                W3.astype(jnp.bfloat16), b3.reshape(E, 1, DIM))
    y = _combine(out, pos0, pos1, g0r, g1r)
    return y.reshape(Bb, Tt, C), aux_a[0, 0]


# speedup vs baseline: 2.1983x; 1.0001x over previous
import functools

import jax
import jax.numpy as jnp
from jax import lax
from jax.experimental import pallas as pl
from jax.experimental.pallas import tpu as pltpu
from jax.experimental.pallas import tpu_sc as plsc

DIM = 1024
E = 8
HID = 4096
S = 4096
BLK_M = 256
NT = 40
N = NT * BLK_M
NW = 32
TOK_PER_W = S // NW
CH = 32


def _shift_down(a, k):
    return jnp.concatenate([jnp.zeros((k,) + a.shape[1:], a.dtype), a[:-k]], axis=0)


def _incl_cumsum0(a):
    acc = a
    k = 1
    while k < a.shape[0]:
        acc = acc + _shift_down(acc, k)
        k *= 2
    return acc


def _router_kernel(xf_ref, wg_ref, bg_ref, aux_ref, mi_ref, g0r_ref, g1r_ref):
    xf = xf_ref[...]
    wg = wg_ref[...]
    logits = lax.dot_general(
        xf, wg, (((1,), (0,)), ((), ())),
        precision=lax.Precision.DEFAULT,
        preferred_element_type=jnp.float32) + bg_ref[...]
    m = jnp.max(logits, axis=1, keepdims=True)
    p = jnp.exp(logits - m)
    probs = p / jnp.sum(p, axis=1, keepdims=True)

    eidx = lax.broadcasted_iota(jnp.int32, (S, E), 1)
    v0 = jnp.max(probs, axis=1, keepdims=True)
    i0 = jnp.min(jnp.where(probs == v0, eidx, E), axis=1, keepdims=True)
    oh0 = (eidx == i0).astype(jnp.float32)
    probs_m = jnp.where(eidx == i0, -1.0, probs)
    v1 = jnp.max(probs_m, axis=1, keepdims=True)
    i1 = jnp.min(jnp.where(probs_m == v1, eidx, E), axis=1, keepdims=True)
    oh1 = (eidx == i1).astype(jnp.float32)

    importance = jnp.mean(probs, axis=0, keepdims=True)
    load = jnp.sum(oh0, axis=0, keepdims=True) / float(S)
    aux_ref[...] = jnp.reshape(float(E) * jnp.sum(importance * load), (1, 1))

    both = oh0 + oh1
    c_in = _incl_cumsum0(both)
    c_ex = c_in - both
    rank0 = jnp.sum(c_ex * oh0, axis=1)
    rank1 = jnp.sum((c_ex + oh0) * oh1, axis=1)

    counts = c_in[-1:, :]
    cap = jnp.ceil(counts / float(BLK_M)) * float(BLK_M)
    base = cap
    k = 1
    while k < E:
        base = base + jnp.concatenate(
            [jnp.zeros((1, k), jnp.float32), base[:, :-k]], axis=1)
        k *= 2
    base_ex = base - cap
    ends = base_ex + cap

    pos0 = jnp.sum(base_ex * oh0, axis=1) + rank0
    pos1 = jnp.sum(base_ex * oh1, axis=1) + rank1

    tstart = lax.broadcasted_iota(jnp.int32, (S, 1), 0).astype(
        jnp.float32) * float(BLK_M)
    te = jnp.sum((ends <= tstart).astype(jnp.float32), axis=1)
    te = jnp.minimum(te, float(E - 1))

    mi_ref[...] = jnp.concatenate(
        [pos0[None, :], pos1[None, :], te[None, :],
         jnp.zeros((5, S), jnp.float32)], axis=0).astype(jnp.int32)
    g0r_ref[...] = jnp.broadcast_to(v0, (S, 16))
    g1r_ref[...] = jnp.broadcast_to(v1, (S, 16))


def _route(xf, Wg, bg):
    return pl.pallas_call(
        _router_kernel,
        out_shape=[
            jax.ShapeDtypeStruct((1, 1), jnp.float32),
            jax.ShapeDtypeStruct((8, S), jnp.int32),
            jax.ShapeDtypeStruct((S, 16), jnp.float32),
            jax.ShapeDtypeStruct((S, 16), jnp.float32),
        ],
    )(xf, Wg, bg.reshape(1, E))


def _gemm_kernel(te_ref, xs_ref, w1_ref, b1_ref, w2_ref, b2_ref, w3_ref,
                 b3_ref, out_ref):
    xb = xs_ref[...].astype(jnp.bfloat16)
    a = lax.dot_general(xb, w1_ref[0], (((1,), (0,)), ((), ())),
                        preferred_element_type=jnp.float32) + b1_ref[0]
    b = lax.dot_general(xb, w2_ref[0], (((1,), (0,)), ((), ())),
                        preferred_element_type=jnp.float32) + b2_ref[0]
    h = (a * jax.nn.sigmoid(a) * b).astype(jnp.bfloat16)
    out_ref[...] = lax.dot_general(
        h, w3_ref[0], (((1,), (0,)), ((), ())),
        preferred_element_type=jnp.float32) + b3_ref[0]


def _gemm(te, xs, W1, b1, W2, b2, W3, b3):
    grid_spec = pltpu.PrefetchScalarGridSpec(
        num_scalar_prefetch=1,
        grid=(NT,),
        in_specs=[
            pl.BlockSpec((BLK_M, DIM), lambda i, te: (i, 0)),
            pl.BlockSpec((1, DIM, HID), lambda i, te: (te[i], 0, 0)),
            pl.BlockSpec((1, 1, HID), lambda i, te: (te[i], 0, 0)),
            pl.BlockSpec((1, DIM, HID), lambda i, te: (te[i], 0, 0)),
            pl.BlockSpec((1, 1, HID), lambda i, te: (te[i], 0, 0)),
            pl.BlockSpec((1, HID, DIM), lambda i, te: (te[i], 0, 0)),
            pl.BlockSpec((1, 1, DIM), lambda i, te: (te[i], 0, 0)),
        ],
        out_specs=pl.BlockSpec((BLK_M, DIM), lambda i, te: (i, 0)),
    )
    return pl.pallas_call(
        _gemm_kernel,
        grid_spec=grid_spec,
        out_shape=jax.ShapeDtypeStruct((N, DIM), jnp.float32),
        compiler_params=pltpu.CompilerParams(
            allow_input_fusion=[False, False, True, False, True, False, True,
                                False]),
    )(te, xs, W1, b1, W2, b2, W3, b3)


def _dispatch_body(xf_hbm, pos0_hbm, pos1_hbm, xs_hbm, rows_v, idx0_v, idx1_v,
                   sem0, sem1):
    wid = lax.axis_index("s") * 2 + lax.axis_index("c")
    for c in range(TOK_PER_W // CH):
        base = wid * TOK_PER_W + c * CH
        pltpu.sync_copy(pos0_hbm.at[pl.ds(base, CH)], idx0_v)
        pltpu.sync_copy(pos1_hbm.at[pl.ds(base, CH)], idx1_v)
        pltpu.sync_copy(xf_hbm.at[pl.ds(base, CH)], rows_v)
        cp0 = pltpu.async_copy(rows_v, xs_hbm.at[idx0_v], sem0)
        cp1 = pltpu.async_copy(rows_v, xs_hbm.at[idx1_v], sem1)
        cp0.wait()
        cp1.wait()


def _combine_body(out_hbm, pos0_hbm, pos1_hbm, g0_hbm, g1_hbm, y_hbm,
                  b0_v, b1_v, y_v, idx0_v, idx1_v, g0_v, g1_v, sem0, sem1):
    wid = lax.axis_index("s") * 2 + lax.axis_index("c")
    for c in range(TOK_PER_W // CH):
        base = wid * TOK_PER_W + c * CH
        pltpu.sync_copy(pos0_hbm.at[pl.ds(base, CH)], idx0_v)
        pltpu.sync_copy(pos1_hbm.at[pl.ds(base, CH)], idx1_v)
        pltpu.sync_copy(g0_hbm.at[pl.ds(base, CH)], g0_v)
        pltpu.sync_copy(g1_hbm.at[pl.ds(base, CH)], g1_v)
        cp0 = pltpu.async_copy(out_hbm.at[idx0_v], b0_v, sem0)
        cp1 = pltpu.async_copy(out_hbm.at[idx1_v], b1_v, sem1)
        cp0.wait()
        cp1.wait()

        def body(r, _):
            g0s = g0_v[r]
            g1s = g1_v[r]
            for j in range(DIM // 16):
                sl = pl.ds(j * 16, 16)
                y_v[r, sl] = g0s * b0_v[r, sl] + g1s * b1_v[r, sl]
            return 0

        lax.fori_loop(0, CH, body, 0)
        pltpu.sync_copy(y_v, y_hbm.at[pl.ds(base, CH)])


@functools.lru_cache(maxsize=1)
def _sc_kernels():
    mesh = plsc.VectorSubcoreMesh(core_axis_name="c", subcore_axis_name="s")
    dispatch = pl.kernel(
        _dispatch_body,
        out_type=jax.ShapeDtypeStruct((N, DIM), jnp.float32),
        mesh=mesh,
        scratch_types=[
            pltpu.VMEM((CH, DIM), jnp.float32),
            pltpu.VMEM((CH,), jnp.int32),
            pltpu.VMEM((CH,), jnp.int32),
            pltpu.SemaphoreType.DMA,
            pltpu.SemaphoreType.DMA,
        ],
    )
    combine = pl.kernel(
        _combine_body,
        out_type=jax.ShapeDtypeStruct((S, DIM), jnp.float32),
        mesh=mesh,
        scratch_types=[
            pltpu.VMEM((CH, DIM), jnp.float32),
            pltpu.VMEM((CH, DIM), jnp.float32),
            pltpu.VMEM((CH, DIM), jnp.float32),
            pltpu.VMEM((CH,), jnp.int32),
            pltpu.VMEM((CH,), jnp.int32),
            pltpu.VMEM((CH, 16), jnp.float32),
            pltpu.VMEM((CH, 16), jnp.float32),
            pltpu.SemaphoreType.DMA,
            pltpu.SemaphoreType.DMA,
        ],
    )
    return dispatch, combine


def kernel(x, Wg, bg, W1, b1, W2, b2, W3, b3):
    Bb, Tt, C = x.shape
    xf = x.reshape(S, C)
    aux_a, mi, g0r, g1r = _route(xf, Wg, bg)
    pos0, pos1, te = mi[0], mi[1], mi[2, :NT]
    _dispatch, _combine = _sc_kernels()
    xs = _dispatch(xf, pos0, pos1)
    out = _gemm(te, xs,
                W1.astype(jnp.bfloat16), b1.reshape(E, 1, HID),
                W2.astype(jnp.bfloat16), b2.reshape(E, 1, HID),
                W3.astype(jnp.bfloat16), b3.reshape(E, 1, DIM))
    y = _combine(out, pos0, pos1, g0r, g1r)
    return y.reshape(Bb, Tt, C), aux_a[0, 0]
